# baseline (speedup 1.0000x reference)
import jax
import jax.numpy as jnp
from jax.experimental import pallas as pl

_GAMMA = 1.5


def _qfocal_body(pred_ref, label_ref, score_ref, out_ref):
    x = pred_ref[...]
    sc = score_ref[...]
    lab = label_ref[...]

    e = jnp.exp(-jnp.abs(x))
    recip = 1.0 / (1.0 + e)
    sig = jnp.where(x >= 0.0, recip, e * recip)
    bce0 = jnp.maximum(x, 0.0) + jnp.log1p(e)
    neg = bce0 * sig * jnp.sqrt(sig)

    s = jnp.max(sc, axis=1, keepdims=True)
    cid = jax.lax.broadcasted_iota(jnp.int32, x.shape, 1)
    m = cid == lab
    xp = jnp.sum(jnp.where(m, x, 0.0), axis=1, keepdims=True)
    ep = jnp.exp(-jnp.abs(xp))
    recp = 1.0 / (1.0 + ep)
    sigp = jnp.where(xp >= 0.0, recp, ep * recp)
    bcep = jnp.maximum(xp, 0.0) - xp * s + jnp.log1p(ep)
    d = jnp.abs(s - sigp)
    pos = bcep * d * jnp.sqrt(d)

    out_ref[...] = jnp.where(m, pos, neg)


def kernel(pred, label, score):
    B, N, C = pred.shape
    M = B * N
    bn = 1024
    p2 = pred.reshape(M, C)
    s2 = score.reshape(M, C)
    l2 = label.reshape(M, 1)
    out = pl.pallas_call(
        _qfocal_body,
        grid=(M // bn,),
        in_specs=[
            pl.BlockSpec((bn, C), lambda i: (i, 0)),
            pl.BlockSpec((bn, 1), lambda i: (i, 0)),
            pl.BlockSpec((bn, C), lambda i: (i, 0)),
        ],
        out_specs=pl.BlockSpec((bn, C), lambda i: (i, 0)),
        out_shape=jax.ShapeDtypeStruct((M, C), jnp.float32),
    )(p2, l2, s2)
    return out.reshape(B, N, C)


# speedup vs baseline: 1.1195x; 1.1195x over previous
import jax
import jax.numpy as jnp
from jax.experimental import pallas as pl

_GAMMA = 1.5


def _qfocal_body(pred_ref, label_ref, score_ref, out_ref):
    x = pred_ref[...]
    sc = score_ref[...]
    lab = label_ref[...]

    e = jnp.exp(-jnp.abs(x))
    recip = 1.0 / (1.0 + e)
    sig = jnp.where(x >= 0.0, recip, e * recip)
    bce0 = jnp.maximum(x, 0.0) + jnp.log1p(e)
    neg = bce0 * sig * jnp.sqrt(sig)

    s = jnp.max(sc, axis=1, keepdims=True)
    d = jnp.abs(s - sig)
    pos = (bce0 - x * s) * d * jnp.sqrt(d)

    cid = jax.lax.broadcasted_iota(jnp.int32, x.shape, 1)
    m = cid == lab
    out_ref[...] = jnp.where(m, pos, neg)


def kernel(pred, label, score):
    B, N, C = pred.shape
    M = B * N
    bn = 1024
    p2 = pred.reshape(M, C)
    s2 = score.reshape(M, C)
    l2 = label.reshape(M, 1)
    out = pl.pallas_call(
        _qfocal_body,
        grid=(M // bn,),
        in_specs=[
            pl.BlockSpec((bn, C), lambda i: (i, 0)),
            pl.BlockSpec((bn, 1), lambda i: (i, 0)),
            pl.BlockSpec((bn, C), lambda i: (i, 0)),
        ],
        out_specs=pl.BlockSpec((bn, C), lambda i: (i, 0)),
        out_shape=jax.ShapeDtypeStruct((M, C), jnp.float32),
    )(p2, l2, s2)
    return out.reshape(B, N, C)


# speedup vs baseline: 1.1441x; 1.0219x over previous
import jax
import jax.numpy as jnp
from jax.experimental import pallas as pl

_GAMMA = 1.5


def _qfocal_body(pred_ref, label_ref, score_ref, out_ref):
    x = pred_ref[0]
    sc = score_ref[0]
    lab = jnp.transpose(label_ref[0])

    e = jnp.exp(-jnp.abs(x))
    recip = 1.0 / (1.0 + e)
    sig = jnp.where(x >= 0.0, recip, e * recip)
    bce0 = jnp.maximum(x, 0.0) + jnp.log1p(e)
    neg = bce0 * sig * jnp.sqrt(sig)

    s = jnp.max(sc, axis=1, keepdims=True)
    d = jnp.abs(s - sig)
    pos = (bce0 - x * s) * d * jnp.sqrt(d)

    cid = jax.lax.broadcasted_iota(jnp.int32, x.shape, 1)
    m = cid == lab
    out_ref[0] = jnp.where(m, pos, neg)


def kernel(pred, label, score):
    B, N, C = pred.shape
    bn = 1024
    nb = N // bn
    l3 = label.reshape(B * nb, 1, bn)
    out = pl.pallas_call(
        _qfocal_body,
        grid=(B, nb),
        in_specs=[
            pl.BlockSpec((1, bn, C), lambda b, i: (b, i, 0)),
            pl.BlockSpec((1, 1, bn), lambda b, i, _nb=nb: (b * _nb + i, 0, 0)),
            pl.BlockSpec((1, bn, C), lambda b, i: (b, i, 0)),
        ],
        out_specs=pl.BlockSpec((1, bn, C), lambda b, i: (b, i, 0)),
        out_shape=jax.ShapeDtypeStruct((B, N, C), jnp.float32),
    )(pred, l3, score)
    return out


# speedup vs baseline: 3.0881x; 2.6992x over previous
import jax
import jax.numpy as jnp
from jax.experimental import pallas as pl

_GAMMA = 1.5


def _qfocal_body(pred_ref, label_ref, score_ref, out_ref):
    x = pred_ref[0]
    sc = score_ref[0]
    lab = label_ref[0]

    e = jnp.exp(-jnp.abs(x))
    recip = 1.0 / (1.0 + e)
    sig = jnp.where(x >= 0.0, recip, e * recip)
    bce0 = jnp.maximum(x, 0.0) + jnp.log1p(e)
    neg = bce0 * sig * jnp.sqrt(sig)

    s = jnp.max(sc, axis=0, keepdims=True)
    d = jnp.abs(s - sig)
    pos = (bce0 - x * s) * d * jnp.sqrt(d)

    cid = jax.lax.broadcasted_iota(jnp.int32, x.shape, 0)
    m = cid == lab
    out_ref[0] = jnp.where(m, pos, neg)


def kernel(pred, label, score):
    B, N, C = pred.shape
    bn = 1024
    nb = N // bn
    pt = jnp.transpose(pred, (0, 2, 1))
    st = jnp.transpose(score, (0, 2, 1))
    l3 = label.reshape(B * nb, 1, bn)
    out = pl.pallas_call(
        _qfocal_body,
        grid=(B, nb),
        in_specs=[
            pl.BlockSpec((1, C, bn), lambda b, i: (b, 0, i)),
            pl.BlockSpec((1, 1, bn), lambda b, i, _nb=nb: (b * _nb + i, 0, 0)),
            pl.BlockSpec((1, C, bn), lambda b, i: (b, 0, i)),
        ],
        out_specs=pl.BlockSpec((1, C, bn), lambda b, i: (b, 0, i)),
        out_shape=jax.ShapeDtypeStruct((B, C, N), jnp.float32),
    )(pt, l3, st)
    return jnp.transpose(out, (0, 2, 1))
